# baseline (speedup 1.0000x reference)
import functools

import jax
import jax.numpy as jnp
from jax import lax
from jax.experimental import pallas as pl
from jax.experimental.pallas import tpu as pltpu
from jax.experimental.pallas import tpu_sc as plsc

NC = 2
NS = 16
NT = NC * NS
LANES = 16
C = 80
CC = 128
HALFN = 5000
TRASH = 120
ACC = HALFN + TRASH

_SC_PARAMS = pltpu.CompilerParams(use_tc_tiling_on_sc=False)


def _mesh():
    return plsc.VectorSubcoreMesh(
        core_axis_name="c", subcore_axis_name="s", num_cores=NC, num_subcores=NS)


def _build_deg(E, NP):
    nchunks = (E // NT) // C
    rows_pt = NP // NS
    ZR = 128
    NZ = rows_pt // ZR

    @functools.partial(
        pl.kernel,
        out_type=jax.ShapeDtypeStruct((NC, NP, LANES), jnp.float32),
        mesh=_mesh(),
        scratch_types=[
            pltpu.VMEM((nchunks, C), jnp.int32),
            pltpu.VMEM((C, LANES), jnp.float32),
            pltpu.VMEM((ZR, LANES), jnp.float32),
            pltpu.VMEM_SHARED((NP, LANES), jnp.float32),
        ],
        compiler_params=_SC_PARAMS,
    )
    def deg_kernel(dst_hbm, out_hbm, didx, ones_v, zb, sh):
        cid = lax.axis_index("c")
        sid = lax.axis_index("s")
        wid = cid * NS + sid

        def fill(i, _):
            ones_v[i] = jnp.full((LANES,), 1.0, jnp.float32)
            return 0
        lax.fori_loop(0, C, fill, 0)

        def fillz(i, _):
            zb[i, :] = jnp.zeros((LANES,), jnp.float32)
            return 0
        lax.fori_loop(0, ZR, fillz, 0)

        base_row = sid * rows_pt
        for j in range(NZ):
            pltpu.sync_copy(zb, sh.at[pl.ds(base_row + j * ZR, ZR)])
        pltpu.sync_copy(dst_hbm.at[wid], didx)
        plsc.subcore_barrier()

        def step(k, _):
            pltpu.sync_copy(ones_v, sh.at[didx.at[k]], add=True)
            return 0
        lax.fori_loop(0, nchunks, step, 0)
        plsc.subcore_barrier()
        pltpu.sync_copy(sh.at[pl.ds(base_row, rows_pt)],
                        out_hbm.at[cid, pl.ds(base_row, rows_pt)])

    return deg_kernel


def _build_prep(E, cap):
    ept = E // NT
    ngr = ept // LANES

    @functools.partial(
        pl.kernel,
        out_type=[
            jax.ShapeDtypeStruct((NT, NC, 2, cap), jnp.int32),
            jax.ShapeDtypeStruct((NT, NC, LANES), jnp.int32),
        ],
        mesh=_mesh(),
        scratch_types=[
            pltpu.VMEM((ept,), jnp.int32),
            pltpu.VMEM((ept,), jnp.int32),
            pltpu.VMEM((cap,), jnp.int32),
            pltpu.VMEM((cap,), jnp.int32),
            pltpu.VMEM((cap,), jnp.int32),
            pltpu.VMEM((cap,), jnp.int32),
            pltpu.VMEM((NC, LANES), jnp.int32),
        ],
        compiler_params=pltpu.CompilerParams(
            use_tc_tiling_on_sc=False, needs_layout_passes=False),
    )
    def prep_kernel(src_hbm, dst_hbm, lists_out, counts_out,
                    srcbuf, dstbuf, lsrc, ldst, hsrc, hdst, cbuf):
        cid = lax.axis_index("c")
        sid = lax.axis_index("s")
        wid = cid * NS + sid
        pltpu.sync_copy(src_hbm.at[wid], srcbuf)
        pltpu.sync_copy(dst_hbm.at[wid], dstbuf)

        def grp(g, carry):
            lo, hi = carry
            sv = srcbuf[pl.ds(g * LANES, LANES)]
            dv = dstbuf[pl.ds(g * LANES, LANES)]
            m = dv < HALFN
            cnt = plsc.all_reduce_population_count(m)[0]
            plsc.store_compressed(lsrc.at[pl.ds(lo, LANES)], sv, mask=m)
            plsc.store_compressed(ldst.at[pl.ds(lo, LANES)], dv, mask=m)
            nm = jnp.logical_not(m)
            plsc.store_compressed(hsrc.at[pl.ds(hi, LANES)], sv, mask=nm)
            plsc.store_compressed(hdst.at[pl.ds(hi, LANES)], dv - HALFN, mask=nm)
            return (lo + cnt, hi + (LANES - cnt))
        lo, hi = lax.fori_loop(0, ngr, grp, (0, 0))

        lanes_i = jnp.arange(LANES, dtype=jnp.int32)

        def pad_dst(dref, pos):
            target = ((pos + CC - 1) // CC) * CC
            rounds = (target - pos + LANES - 1) // LANES

            def pr(r, _):
                dref[pl.ds(pos + r * LANES, LANES)] = (
                    HALFN + (r % 6) * LANES + lanes_i)
                return 0
            lax.fori_loop(0, rounds, pr, 0)
            return target

        def pad_src(sref, pos):
            target = ((pos + CC - 1) // CC) * CC
            rounds = (target - pos + LANES - 1) // LANES

            def pr(r, _):
                sref[pl.ds(pos + r * LANES, LANES)] = jnp.zeros((LANES,), jnp.int32)
                return 0
            lax.fori_loop(0, rounds, pr, 0)

        pad_src(lsrc, lo)
        pad_src(hsrc, hi)
        lo_t = pad_dst(ldst, lo)
        hi_t = pad_dst(hdst, hi)

        cbuf[0] = jnp.full((LANES,), 1, jnp.int32) * lo_t
        cbuf[1] = jnp.full((LANES,), 1, jnp.int32) * hi_t
        pltpu.sync_copy(lsrc, lists_out.at[wid, 0, 0])
        pltpu.sync_copy(ldst, lists_out.at[wid, 0, 1])
        pltpu.sync_copy(hsrc, lists_out.at[wid, 1, 0])
        pltpu.sync_copy(hdst, lists_out.at[wid, 1, 1])
        pltpu.sync_copy(cbuf, counts_out.at[wid])

    return prep_kernel


def _build_scatter(N, HD, cap):
    rows_pt = ACC // NS
    ZR = 64
    NZ = rows_pt // ZR
    nchmax = cap // CC

    @functools.partial(
        pl.kernel,
        out_type=jax.ShapeDtypeStruct((NC, ACC, HD), jnp.float32),
        mesh=_mesh(),
        scratch_types=[
            pltpu.VMEM((2, nchmax, CC), jnp.int32),
            pltpu.VMEM((2, nchmax, CC), jnp.int32),
            pltpu.VMEM((2, LANES), jnp.int32),
            pltpu.VMEM((CC, HD), jnp.float32),
            pltpu.VMEM((ZR, HD), jnp.float32),
            pltpu.VMEM_SHARED((ACC, HD), jnp.float32),
            pltpu.SemaphoreType.DMA,
        ],
        compiler_params=_SC_PARAMS,
    )
    def scat_kernel(hwp_hbm, lists_hbm, counts_hbm, out_hbm,
                    sidx, didx, cbuf, rows, zb, sh, sem):
        cid = lax.axis_index("c")
        sid = lax.axis_index("s")

        def fillz(i, _):
            for j in range(HD // LANES):
                zb[i, pl.ds(j * LANES, LANES)] = jnp.zeros((LANES,), jnp.float32)
            return 0
        lax.fori_loop(0, ZR, fillz, 0)

        base_row = sid * rows_pt
        for j in range(NZ):
            pltpu.sync_copy(zb, sh.at[pl.ds(base_row + j * ZR, ZR)])
        for j in range(2):
            pt = 2 * sid + j
            pltpu.sync_copy(lists_hbm.at[pt, cid, 0], sidx.at[j])
            pltpu.sync_copy(lists_hbm.at[pt, cid, 1], didx.at[j])
            pltpu.sync_copy(counts_hbm.at[pt, cid], cbuf.at[j])
        plsc.subcore_barrier()

        for j in range(2):
            nch = cbuf[j][0] // CC

            def step(k, _):
                pltpu.async_copy(hwp_hbm.at[sidx.at[j, k]], rows, sem).wait()
                pltpu.sync_copy(rows, sh.at[didx.at[j, k]], add=True)
                return 0
            lax.fori_loop(0, nch, step, 0)
        plsc.subcore_barrier()
        pltpu.sync_copy(sh.at[pl.ds(base_row, rows_pt)],
                        out_hbm.at[cid, pl.ds(base_row, rows_pt)])

    return scat_kernel


def _build_pool(N, HD, B):
    P = 320

    @functools.partial(
        pl.kernel,
        out_type=jax.ShapeDtypeStruct((NT, B, HD), jnp.float32),
        mesh=_mesh(),
        scratch_types=[
            pltpu.VMEM((P, HD), jnp.float32),
            pltpu.VMEM((P,), jnp.int32),
            pltpu.VMEM((B, HD), jnp.float32),
        ],
    )
    def pool_kernel(h_hbm, batch_hbm, out_hbm, hv, bv, acc):
        cid = lax.axis_index("c")
        sid = lax.axis_index("s")
        wid = cid * NS + sid
        base = jnp.minimum(wid * P, N - P)

        def filln(i, _):
            for j in range(HD // LANES):
                acc[i, pl.ds(j * LANES, LANES)] = jnp.full(
                    (LANES,), -jnp.inf, jnp.float32)
            return 0
        lax.fori_loop(0, B, filln, 0)

        pltpu.sync_copy(h_hbm.at[pl.ds(base, P)], hv)
        pltpu.sync_copy(batch_hbm.at[pl.ds(base, P)], bv)

        def group(gi, _):
            bvec = bv[pl.ds(gi * LANES, LANES)]
            for l in range(LANES):
                b = bvec[l]
                for j in range(HD // LANES):
                    sl = pl.ds(j * LANES, LANES)
                    acc[b, sl] = jnp.maximum(acc[b, sl], hv[gi * LANES + l, sl])
            return 0
        lax.fori_loop(0, P // LANES, group, 0)
        pltpu.sync_copy(acc, out_hbm.at[wid])

    return pool_kernel


def _k0(x, W, degp, RB=1000):
    N, D = x.shape
    HD = W.shape[1]

    def body(x_ref, w_ref, degp_ref, hwp_ref, dis_ref):
        deg = degp_ref[0, :, 0:1] + degp_ref[1, :, 0:1] + 1.0
        dis = jnp.where(deg > 0, lax.rsqrt(jnp.maximum(deg, 1e-12)), 0.0)
        hw = jnp.dot(x_ref[...], w_ref[...], preferred_element_type=jnp.float32)
        hwp_ref[...] = hw * dis
        dis_ref[...] = dis

    return pl.pallas_call(
        body,
        grid=(N // RB,),
        in_specs=[
            pl.BlockSpec((RB, D), lambda i: (i, 0)),
            pl.BlockSpec((D, HD), lambda i: (0, 0)),
            pl.BlockSpec((NC, RB, LANES), lambda i: (0, i, 0)),
        ],
        out_specs=[
            pl.BlockSpec((RB, HD), lambda i: (i, 0)),
            pl.BlockSpec((RB, 1), lambda i: (i, 0)),
        ],
        out_shape=[
            jax.ShapeDtypeStruct((N, HD), jnp.float32),
            jax.ShapeDtypeStruct((N, 1), jnp.float32),
        ],
    )(x, W, degp)


def _kc(parts, hwp, dis, b, W, RB=1000):
    N, HD = hwp.shape
    nb = HALFN // RB

    def body(part_ref, hwp_ref, dis_ref, b_ref, w_ref, out_ref):
        s = part_ref[0] + hwp_ref[...]
        h = jnp.maximum(s * dis_ref[...] + b_ref[...], 0.0)
        out_ref[...] = jnp.dot(
            h, w_ref[...], preferred_element_type=jnp.float32) * dis_ref[...]

    return pl.pallas_call(
        body,
        grid=(N // RB,),
        in_specs=[
            pl.BlockSpec((1, RB, HD), lambda i: (i // nb, i % nb, 0)),
            pl.BlockSpec((RB, HD), lambda i: (i, 0)),
            pl.BlockSpec((RB, 1), lambda i: (i, 0)),
            pl.BlockSpec((1, HD), lambda i: (0, 0)),
            pl.BlockSpec((HD, HD), lambda i: (0, 0)),
        ],
        out_specs=pl.BlockSpec((RB, HD), lambda i: (i, 0)),
        out_shape=jax.ShapeDtypeStruct((N, HD), jnp.float32),
    )(parts, hwp, dis, b, W)


def _kc_final(parts, hwp, dis, b, RB=1000):
    N, HD = hwp.shape
    nb = HALFN // RB

    def body(part_ref, hwp_ref, dis_ref, b_ref, out_ref):
        s = part_ref[0] + hwp_ref[...]
        out_ref[...] = jnp.maximum(s * dis_ref[...] + b_ref[...], 0.0)

    return pl.pallas_call(
        body,
        grid=(N // RB,),
        in_specs=[
            pl.BlockSpec((1, RB, HD), lambda i: (i // nb, i % nb, 0)),
            pl.BlockSpec((RB, HD), lambda i: (i, 0)),
            pl.BlockSpec((RB, 1), lambda i: (i, 0)),
            pl.BlockSpec((1, HD), lambda i: (0, 0)),
        ],
        out_specs=pl.BlockSpec((RB, HD), lambda i: (i, 0)),
        out_shape=jax.ShapeDtypeStruct((N, HD), jnp.float32),
    )(parts, hwp, dis, b)


def _head(gp, fr, ad, W_r1, b_r1, W_r2, b_r2, W_out, b_out):
    B = fr.shape[0]
    PROP = W_out.shape[1]

    def body(gp_ref, fr_ref, ad_ref, wr1, br1, wr2, br2, wo, bo, out_ref):
        g = jnp.max(gp_ref[...], axis=0)
        z = jnp.concatenate([g, fr_ref[...], ad_ref[...]], axis=1)
        z1 = jnp.dot(z, wr1[...], preferred_element_type=jnp.float32) + br1[...]
        s = z1 * jax.nn.sigmoid(z1)
        z = z + jnp.dot(s, wr2[...], preferred_element_type=jnp.float32) + br2[...]
        out_ref[...] = jnp.dot(
            z, wo[...], preferred_element_type=jnp.float32) + bo[...]

    return pl.pallas_call(
        body,
        out_shape=jax.ShapeDtypeStruct((B, PROP), jnp.float32),
    )(gp, fr, ad, W_r1, b_r1, W_r2, b_r2, W_out, b_out)


def kernel(x, edge_index, batch, frag_levels, adduct_feats,
           W_in, b_in, W_mid, b_mid, W_r1, b_r1, W_r2, b_r2, W_out, b_out):
    N, D = x.shape
    HD = W_in.shape[1]
    E = edge_index.shape[1]
    B = frag_levels.shape[0] // 8

    src32 = edge_index[0].astype(jnp.int32)
    dst32 = edge_index[1].astype(jnp.int32)
    ept = E // NT
    dst_r32 = dst32.reshape(NT, ept // C, C)
    batch32 = batch.astype(jnp.int32)
    fr = frag_levels.reshape(B, 8)
    ad = adduct_feats.reshape(B, 8)

    cap = (-(-(ept + CC) // CC)) * CC
    NP = NS * 640
    deg_call = _build_deg(E, NP)
    prep_call = _build_prep(E, cap)
    scat_call = _build_scatter(N, HD, cap)
    pool_call = _build_pool(N, HD, B)

    degp = deg_call(dst_r32)
    lists, counts = prep_call(src32.reshape(NT, ept), dst32.reshape(NT, ept))
    lists = lists.reshape(NT, NC, 2, cap // CC, CC)
    hwp, dis = _k0(x, W_in, degp)

    biases = [b_in.reshape(1, HD)] + [b_mid[i].reshape(1, HD) for i in range(W_mid.shape[0])]
    Ws = [W_mid[i] for i in range(W_mid.shape[0])]
    nlayers = 1 + W_mid.shape[0]

    h = None
    for li in range(nlayers):
        parts = scat_call(hwp, lists, counts)
        if li < nlayers - 1:
            hwp = _kc(parts, hwp, dis, biases[li], Ws[li])
        else:
            h = _kc_final(parts, hwp, dis, biases[li])

    gp = pool_call(h, batch32)
    return _head(gp, fr, ad, W_r1, b_r1.reshape(1, HD), W_r2,
                 b_r2.reshape(1, HD + 16), W_out, b_out.reshape(1, W_out.shape[1]))


# speedup vs baseline: 1.2397x; 1.2397x over previous
import functools

import jax
import jax.numpy as jnp
from jax import lax
from jax.experimental import pallas as pl
from jax.experimental.pallas import tpu as pltpu
from jax.experimental.pallas import tpu_sc as plsc

NC = 2
NS = 16
NT = NC * NS
LANES = 16
C = 80
CC = 128
HALFN = 5000
TRASH = 120
ACC = HALFN + TRASH

_SC_PARAMS = pltpu.CompilerParams(use_tc_tiling_on_sc=False)


def _mesh():
    return plsc.VectorSubcoreMesh(
        core_axis_name="c", subcore_axis_name="s", num_cores=NC, num_subcores=NS)


def _build_deg(E, NP):
    nchunks = (E // NT) // C
    rows_pt = NP // NS
    ZR = 128
    NZ = rows_pt // ZR

    @functools.partial(
        pl.kernel,
        out_type=jax.ShapeDtypeStruct((NC, NP, LANES), jnp.float32),
        mesh=_mesh(),
        scratch_types=[
            pltpu.VMEM((nchunks, C), jnp.int32),
            pltpu.VMEM((C, LANES), jnp.float32),
            pltpu.VMEM((ZR, LANES), jnp.float32),
            pltpu.VMEM_SHARED((NP, LANES), jnp.float32),
        ],
        compiler_params=_SC_PARAMS,
    )
    def deg_kernel(dst_hbm, out_hbm, didx, ones_v, zb, sh):
        cid = lax.axis_index("c")
        sid = lax.axis_index("s")
        wid = cid * NS + sid

        def fill(i, _):
            ones_v[i] = jnp.full((LANES,), 1.0, jnp.float32)
            return 0
        lax.fori_loop(0, C, fill, 0)

        def fillz(i, _):
            zb[i, :] = jnp.zeros((LANES,), jnp.float32)
            return 0
        lax.fori_loop(0, ZR, fillz, 0)

        base_row = sid * rows_pt
        for j in range(NZ):
            pltpu.sync_copy(zb, sh.at[pl.ds(base_row + j * ZR, ZR)])
        pltpu.sync_copy(dst_hbm.at[wid], didx)
        plsc.subcore_barrier()

        def step(k, _):
            pltpu.sync_copy(ones_v, sh.at[didx.at[k]], add=True)
            return 0
        lax.fori_loop(0, nchunks, step, 0)
        plsc.subcore_barrier()
        pltpu.sync_copy(sh.at[pl.ds(base_row, rows_pt)],
                        out_hbm.at[cid, pl.ds(base_row, rows_pt)])

    return deg_kernel


def _build_prep(E, cap):
    ept = E // NT
    ngr = ept // LANES

    @functools.partial(
        pl.kernel,
        out_type=[
            jax.ShapeDtypeStruct((NT, NC, 2, cap), jnp.int32),
            jax.ShapeDtypeStruct((NT, NC, LANES), jnp.int32),
        ],
        mesh=_mesh(),
        scratch_types=[
            pltpu.VMEM((ept,), jnp.int32),
            pltpu.VMEM((ept,), jnp.int32),
            pltpu.VMEM((cap,), jnp.int32),
            pltpu.VMEM((cap,), jnp.int32),
            pltpu.VMEM((cap,), jnp.int32),
            pltpu.VMEM((cap,), jnp.int32),
            pltpu.VMEM((NC, LANES), jnp.int32),
        ],
        compiler_params=pltpu.CompilerParams(
            use_tc_tiling_on_sc=False, needs_layout_passes=False),
    )
    def prep_kernel(src_hbm, dst_hbm, lists_out, counts_out,
                    srcbuf, dstbuf, lsrc, ldst, hsrc, hdst, cbuf):
        cid = lax.axis_index("c")
        sid = lax.axis_index("s")
        wid = cid * NS + sid
        pltpu.sync_copy(src_hbm.at[wid], srcbuf)
        pltpu.sync_copy(dst_hbm.at[wid], dstbuf)

        def grp(g, carry):
            lo, hi = carry
            sv = srcbuf[pl.ds(g * LANES, LANES)]
            dv = dstbuf[pl.ds(g * LANES, LANES)]
            m = dv < HALFN
            cnt = plsc.all_reduce_population_count(m)[0]
            plsc.store_compressed(lsrc.at[pl.ds(lo, LANES)], sv, mask=m)
            plsc.store_compressed(ldst.at[pl.ds(lo, LANES)], dv, mask=m)
            nm = jnp.logical_not(m)
            plsc.store_compressed(hsrc.at[pl.ds(hi, LANES)], sv, mask=nm)
            plsc.store_compressed(hdst.at[pl.ds(hi, LANES)], dv - HALFN, mask=nm)
            return (lo + cnt, hi + (LANES - cnt))
        lo, hi = lax.fori_loop(0, ngr, grp, (0, 0))

        lanes_i = jnp.arange(LANES, dtype=jnp.int32)

        def pad_dst(dref, pos):
            target = ((pos + CC - 1) // CC) * CC
            rounds = (target - pos + LANES - 1) // LANES

            def pr(r, _):
                dref[pl.ds(pos + r * LANES, LANES)] = (
                    HALFN + (r % 6) * LANES + lanes_i)
                return 0
            lax.fori_loop(0, rounds, pr, 0)
            return target

        def pad_src(sref, pos):
            target = ((pos + CC - 1) // CC) * CC
            rounds = (target - pos + LANES - 1) // LANES

            def pr(r, _):
                sref[pl.ds(pos + r * LANES, LANES)] = jnp.zeros((LANES,), jnp.int32)
                return 0
            lax.fori_loop(0, rounds, pr, 0)

        pad_src(lsrc, lo)
        pad_src(hsrc, hi)
        lo_t = pad_dst(ldst, lo)
        hi_t = pad_dst(hdst, hi)

        cbuf[0] = jnp.full((LANES,), 1, jnp.int32) * lo_t
        cbuf[1] = jnp.full((LANES,), 1, jnp.int32) * hi_t
        pltpu.sync_copy(lsrc, lists_out.at[wid, 0, 0])
        pltpu.sync_copy(ldst, lists_out.at[wid, 0, 1])
        pltpu.sync_copy(hsrc, lists_out.at[wid, 1, 0])
        pltpu.sync_copy(hdst, lists_out.at[wid, 1, 1])
        pltpu.sync_copy(cbuf, counts_out.at[wid])

    return prep_kernel


def _build_scatter(N, HD, cap):
    rows_pt = ACC // NS
    ZR = 16
    NZ = rows_pt // ZR
    nchmax = cap // CC

    @functools.partial(
        pl.kernel,
        out_type=jax.ShapeDtypeStruct((NC, ACC, HD), jnp.float32),
        mesh=_mesh(),
        scratch_types=[
            pltpu.VMEM((nchmax, CC), jnp.int32),
            pltpu.VMEM((nchmax, CC), jnp.int32),
            pltpu.VMEM((2, LANES), jnp.int32),
            pltpu.VMEM((4, CC, HD), jnp.float32),
            pltpu.VMEM((ZR, HD), jnp.float32),
            pltpu.VMEM_SHARED((ACC, HD), jnp.float32),
            pltpu.SemaphoreType.DMA((4,)),
            pltpu.SemaphoreType.DMA((4,)),
        ],
        compiler_params=_SC_PARAMS,
    )
    def scat_kernel(hwp_hbm, lists_hbm, counts_hbm, out_hbm,
                    sidx, didx, cbuf, rows, zb, sh, gsem, ssem):
        cid = lax.axis_index("c")
        sid = lax.axis_index("s")

        def fillz(i, _):
            for j in range(HD // LANES):
                zb[i, pl.ds(j * LANES, LANES)] = jnp.zeros((LANES,), jnp.float32)
            return 0
        lax.fori_loop(0, ZR, fillz, 0)

        base_row = sid * rows_pt
        for j in range(NZ):
            pltpu.sync_copy(zb, sh.at[pl.ds(base_row + j * ZR, ZR)])
        for j in range(2):
            pltpu.sync_copy(counts_hbm.at[2 * sid + j, cid], cbuf.at[j])
        plsc.subcore_barrier()

        NBUF = 4
        L = 2
        for j in range(2):
            pt = 2 * sid + j
            pltpu.sync_copy(lists_hbm.at[pt, cid, 0], sidx)
            pltpu.sync_copy(lists_hbm.at[pt, cid, 1], didx)
            nch = cbuf[j][0] // CC

            def gissue(k, b):
                pltpu.async_copy(hwp_hbm.at[sidx.at[k]], rows.at[b], gsem.at[b])

            def gwait(b):
                pltpu.make_async_copy(
                    hwp_hbm.at[sidx.at[0]], rows.at[b], gsem.at[b]).wait()

            def sissue(k, b):
                pltpu.async_copy(rows.at[b], sh.at[didx.at[k]], ssem.at[b], add=True)

            def swait(b):
                pltpu.make_async_copy(
                    rows.at[b], sh.at[didx.at[0]], ssem.at[b]).wait()

            for b in range(L):
                @pl.when(b < nch)
                def _(b=b):
                    gissue(b, b)

            def ring(r, _):
                for b in range(NBUF):
                    k = r * NBUF + b
                    kn = k + L
                    bn = (b + L) % NBUF

                    @pl.when(kn < nch)
                    def _(k=k, kn=kn, bn=bn):
                        @pl.when(kn >= NBUF)
                        def _():
                            swait(bn)
                        gissue(kn, bn)

                    @pl.when(k < nch)
                    def _(k=k, b=b):
                        gwait(b)
                        sissue(k, b)
                return 0
            lax.fori_loop(0, (nch + NBUF - 1) // NBUF, ring, 0)

            for b in range(NBUF):
                @pl.when(b < nch)
                def _(b=b):
                    swait(b)
        plsc.subcore_barrier()
        pltpu.sync_copy(sh.at[pl.ds(base_row, rows_pt)],
                        out_hbm.at[cid, pl.ds(base_row, rows_pt)])

    return scat_kernel


def _build_pool(N, HD, B):
    P = 320

    @functools.partial(
        pl.kernel,
        out_type=jax.ShapeDtypeStruct((NT, B, HD), jnp.float32),
        mesh=_mesh(),
        scratch_types=[
            pltpu.VMEM((P, HD), jnp.float32),
            pltpu.VMEM((P,), jnp.int32),
            pltpu.VMEM((B, HD), jnp.float32),
        ],
    )
    def pool_kernel(h_hbm, batch_hbm, out_hbm, hv, bv, acc):
        cid = lax.axis_index("c")
        sid = lax.axis_index("s")
        wid = cid * NS + sid
        base = jnp.minimum(wid * P, N - P)

        def filln(i, _):
            for j in range(HD // LANES):
                acc[i, pl.ds(j * LANES, LANES)] = jnp.full(
                    (LANES,), -jnp.inf, jnp.float32)
            return 0
        lax.fori_loop(0, B, filln, 0)

        pltpu.sync_copy(h_hbm.at[pl.ds(base, P)], hv)
        pltpu.sync_copy(batch_hbm.at[pl.ds(base, P)], bv)

        def group(gi, _):
            bvec = bv[pl.ds(gi * LANES, LANES)]
            for l in range(LANES):
                b = bvec[l]
                for j in range(HD // LANES):
                    sl = pl.ds(j * LANES, LANES)
                    acc[b, sl] = jnp.maximum(acc[b, sl], hv[gi * LANES + l, sl])
            return 0
        lax.fori_loop(0, P // LANES, group, 0)
        pltpu.sync_copy(acc, out_hbm.at[wid])

    return pool_kernel


def _k0(x, W, degp, RB=1000):
    N, D = x.shape
    HD = W.shape[1]

    def body(x_ref, w_ref, degp_ref, hwp_ref, dis_ref):
        deg = degp_ref[0, :, 0:1] + degp_ref[1, :, 0:1] + 1.0
        dis = jnp.where(deg > 0, lax.rsqrt(jnp.maximum(deg, 1e-12)), 0.0)
        hw = jnp.dot(x_ref[...], w_ref[...], preferred_element_type=jnp.float32)
        hwp_ref[...] = hw * dis
        dis_ref[...] = dis

    return pl.pallas_call(
        body,
        grid=(N // RB,),
        in_specs=[
            pl.BlockSpec((RB, D), lambda i: (i, 0)),
            pl.BlockSpec((D, HD), lambda i: (0, 0)),
            pl.BlockSpec((NC, RB, LANES), lambda i: (0, i, 0)),
        ],
        out_specs=[
            pl.BlockSpec((RB, HD), lambda i: (i, 0)),
            pl.BlockSpec((RB, 1), lambda i: (i, 0)),
        ],
        out_shape=[
            jax.ShapeDtypeStruct((N, HD), jnp.float32),
            jax.ShapeDtypeStruct((N, 1), jnp.float32),
        ],
    )(x, W, degp)


def _kc(parts, hwp, dis, b, W, RB=1000):
    N, HD = hwp.shape
    nb = HALFN // RB

    def body(part_ref, hwp_ref, dis_ref, b_ref, w_ref, out_ref):
        s = part_ref[0] + hwp_ref[...]
        h = jnp.maximum(s * dis_ref[...] + b_ref[...], 0.0)
        out_ref[...] = jnp.dot(
            h, w_ref[...], preferred_element_type=jnp.float32) * dis_ref[...]

    return pl.pallas_call(
        body,
        grid=(N // RB,),
        in_specs=[
            pl.BlockSpec((1, RB, HD), lambda i: (i // nb, i % nb, 0)),
            pl.BlockSpec((RB, HD), lambda i: (i, 0)),
            pl.BlockSpec((RB, 1), lambda i: (i, 0)),
            pl.BlockSpec((1, HD), lambda i: (0, 0)),
            pl.BlockSpec((HD, HD), lambda i: (0, 0)),
        ],
        out_specs=pl.BlockSpec((RB, HD), lambda i: (i, 0)),
        out_shape=jax.ShapeDtypeStruct((N, HD), jnp.float32),
    )(parts, hwp, dis, b, W)


def _kc_final(parts, hwp, dis, b, RB=1000):
    N, HD = hwp.shape
    nb = HALFN // RB

    def body(part_ref, hwp_ref, dis_ref, b_ref, out_ref):
        s = part_ref[0] + hwp_ref[...]
        out_ref[...] = jnp.maximum(s * dis_ref[...] + b_ref[...], 0.0)

    return pl.pallas_call(
        body,
        grid=(N // RB,),
        in_specs=[
            pl.BlockSpec((1, RB, HD), lambda i: (i // nb, i % nb, 0)),
            pl.BlockSpec((RB, HD), lambda i: (i, 0)),
            pl.BlockSpec((RB, 1), lambda i: (i, 0)),
            pl.BlockSpec((1, HD), lambda i: (0, 0)),
        ],
        out_specs=pl.BlockSpec((RB, HD), lambda i: (i, 0)),
        out_shape=jax.ShapeDtypeStruct((N, HD), jnp.float32),
    )(parts, hwp, dis, b)


def _head(gp, fr, ad, W_r1, b_r1, W_r2, b_r2, W_out, b_out):
    B = fr.shape[0]
    PROP = W_out.shape[1]

    def body(gp_ref, fr_ref, ad_ref, wr1, br1, wr2, br2, wo, bo, out_ref):
        g = jnp.max(gp_ref[...], axis=0)
        z = jnp.concatenate([g, fr_ref[...], ad_ref[...]], axis=1)
        z1 = jnp.dot(z, wr1[...], preferred_element_type=jnp.float32) + br1[...]
        s = z1 * jax.nn.sigmoid(z1)
        z = z + jnp.dot(s, wr2[...], preferred_element_type=jnp.float32) + br2[...]
        out_ref[...] = jnp.dot(
            z, wo[...], preferred_element_type=jnp.float32) + bo[...]

    return pl.pallas_call(
        body,
        out_shape=jax.ShapeDtypeStruct((B, PROP), jnp.float32),
    )(gp, fr, ad, W_r1, b_r1, W_r2, b_r2, W_out, b_out)


def kernel(x, edge_index, batch, frag_levels, adduct_feats,
           W_in, b_in, W_mid, b_mid, W_r1, b_r1, W_r2, b_r2, W_out, b_out):
    N, D = x.shape
    HD = W_in.shape[1]
    E = edge_index.shape[1]
    B = frag_levels.shape[0] // 8

    src32 = edge_index[0].astype(jnp.int32)
    dst32 = edge_index[1].astype(jnp.int32)
    ept = E // NT
    dst_r32 = dst32.reshape(NT, ept // C, C)
    batch32 = batch.astype(jnp.int32)
    fr = frag_levels.reshape(B, 8)
    ad = adduct_feats.reshape(B, 8)

    cap = (-(-(ept + CC) // CC)) * CC
    NP = NS * 640
    deg_call = _build_deg(E, NP)
    prep_call = _build_prep(E, cap)
    scat_call = _build_scatter(N, HD, cap)
    pool_call = _build_pool(N, HD, B)

    degp = deg_call(dst_r32)
    lists, counts = prep_call(src32.reshape(NT, ept), dst32.reshape(NT, ept))
    lists = lists.reshape(NT, NC, 2, cap // CC, CC)
    hwp, dis = _k0(x, W_in, degp)

    biases = [b_in.reshape(1, HD)] + [b_mid[i].reshape(1, HD) for i in range(W_mid.shape[0])]
    Ws = [W_mid[i] for i in range(W_mid.shape[0])]
    nlayers = 1 + W_mid.shape[0]

    h = None
    for li in range(nlayers):
        parts = scat_call(hwp, lists, counts)
        if li < nlayers - 1:
            hwp = _kc(parts, hwp, dis, biases[li], Ws[li])
        else:
            h = _kc_final(parts, hwp, dis, biases[li])

    gp = pool_call(h, batch32)
    return _head(gp, fr, ad, W_r1, b_r1.reshape(1, HD), W_r2,
                 b_r2.reshape(1, HD + 16), W_out, b_out.reshape(1, W_out.shape[1]))
